# baseline (speedup 1.0000x reference)
import jax
import jax.numpy as jnp
from jax.experimental import pallas as pl


def _inorm(x, eps=1e-5):
    mean = jnp.mean(x, axis=0, keepdims=True)
    var = jnp.var(x, axis=0, keepdims=True)
    return (x - mean) / jnp.sqrt(var + eps)


def _cheb(x, row, col, ew, W, b, n):
    deg = jax.ops.segment_sum(ew, row, num_segments=n)
    dis = jnp.where(deg > 0, 1.0 / jnp.sqrt(deg), 0.0)
    lw = -dis[row] * ew * dis[col]
    def Lmul(v):
        return jax.ops.segment_sum(lw[:, None] * v[row], col, num_segments=n)
    Tx0 = x
    Tx1 = Lmul(x)
    out = Tx0 @ W[0] + Tx1 @ W[1]
    Tx2 = 2.0 * Lmul(Tx1) - Tx0
    return out + Tx2 @ W[2] + b


def _edge_mlp(src, dest, ea, W1, b1, g, be, W2, b2, eps=1e-5):
    h = jnp.concatenate([src, dest, ea], axis=1)
    h = jax.nn.relu(h @ W1 + b1)
    mu = jnp.mean(h, axis=-1, keepdims=True)
    var = jnp.var(h, axis=-1, keepdims=True)
    h = (h - mu) / jnp.sqrt(var + eps) * g + be
    return h @ W2 + b2


def _mlp_body(xs_ref, w1_ref, b1_ref, w2_ref, b2_ref, w3_ref, b3_ref, out_ref):
    h = jnp.maximum(xs_ref[...] @ w1_ref[...] + b1_ref[...], 0.0)
    h = jnp.maximum(h @ w2_ref[...] + b2_ref[...], 0.0)
    out_ref[...] = jnp.maximum(h @ w3_ref[...] + b3_ref[...], 0.0)


def _node_mlp(xs, lin1_W, lin1_b, lin2_W, lin2_b, lin3_W, lin3_b):
    n = xs.shape[0]
    blk = 2000
    grid = n // blk
    full = lambda s: pl.BlockSpec(s, lambda i: (0,) * len(s))
    return pl.pallas_call(
        _mlp_body,
        grid=(grid,),
        in_specs=[
            pl.BlockSpec((blk, 85), lambda i: (i, 0)),
            full((85, 40)), full((40,)),
            full((40, 16)), full((16,)),
            full((16, 1)), full((1,)),
        ],
        out_specs=pl.BlockSpec((blk, 1), lambda i: (i, 0)),
        out_shape=jax.ShapeDtypeStruct((n, 1), jnp.float32),
    )(xs, lin1_W, lin1_b, lin2_W, lin2_b, lin3_W, lin3_b)


def kernel(x, edge_index, edge_attr, nc1_W, nc1_b, nc2_W, nc2_b, nc3_W, nc3_b,
           nc4_W, nc4_b, lin1_W, lin1_b, lin2_W, lin2_b, lin3_W, lin3_b,
           ec1_W1, ec1_b1, ec1_g, ec1_be, ec1_W2, ec1_b2,
           ec2_W1, ec2_b1, ec2_g, ec2_be, ec2_W2, ec2_b2):
    x = x.reshape(-1, 1)
    n = x.shape[0]
    row = edge_index[0]
    col = edge_index[1]
    ew = edge_attr.reshape(-1)
    x1 = jax.nn.relu(_cheb(_inorm(x), row, col, ew, nc1_W, nc1_b, n))
    x2 = jax.nn.relu(_cheb(_inorm(x1), row, col, ew, nc2_W, nc2_b, n))
    x3 = jax.nn.relu(_cheb(_inorm(x2), row, col, ew, nc3_W, nc3_b, n))
    x4 = jax.nn.relu(_cheb(_inorm(x3), row, col, ew, nc4_W, nc4_b, n))
    xs = jnp.concatenate([x1, x2, x3, x4], axis=1)
    xo = _node_mlp(xs, lin1_W, lin1_b, lin2_W, lin2_b, lin3_W, lin3_b)
    ea = jax.nn.relu(_edge_mlp(xo[row], xo[col], edge_attr,
                               ec1_W1, ec1_b1, ec1_g, ec1_be, ec1_W2, ec1_b2))
    ea = jax.nn.relu(_edge_mlp(xo[row], xo[col], ea,
                               ec2_W1, ec2_b1, ec2_g, ec2_be, ec2_W2, ec2_b2))
    return (xo, ea)


# speedup vs baseline: 4.5175x; 4.5175x over previous
import functools
import jax
import jax.numpy as jnp
from jax import lax
from jax.experimental import pallas as pl
from jax.experimental.pallas import tpu as pltpu
from jax.experimental.pallas import tpu_sc as plsc

_N = 50000
_E = 800000
_NW = 32
_NPAD = 51200
_RPT = _NPAD // 16
_EPAD = 819200
_EP = _EPAD // _NW
_CH = 128
_NCH = 8
_BLK = _CH * _NCH
_NBLK = _EP // _BLK

_mesh = plsc.VectorSubcoreMesh(core_axis_name="c", subcore_axis_name="s")


def _seg_body(k, gather, table, row2d, col2d, ew2d, zeros, out,
              ridx, cidx, eww, fidx, gath, val, tbl, acc, gsem, ssem):
    core = lax.axis_index("c")
    tid = lax.axis_index("s")
    wid = core * 16 + tid
    sl = pl.ds(tid * k * _RPT, k * _RPT)
    pltpu.sync_copy(zeros.at[sl], acc.at[sl])
    if gather:
        pltpu.sync_copy(table.at[sl], tbl.at[sl])
    plsc.subcore_barrier()

    @pl.loop(0, _NBLK)
    def _blk(b):
        rb = wid * (_EP // _CH) + b * _NCH
        pltpu.sync_copy(col2d.at[pl.ds(rb, _NCH)], cidx)
        pltpu.sync_copy(ew2d.at[pl.ds(rb, _NCH)], eww)
        if gather:
            pltpu.sync_copy(row2d.at[pl.ds(rb, _NCH)], ridx)
        for f in range(k):
            if gather:
                if k > 1:
                    @pl.loop(0, _CH, step=16)
                    def _ofs(i):
                        for j in range(_NCH):
                            fidx[j, pl.ds(i, 16)] = (
                                ridx[j, pl.ds(i, 16)] + f * _NPAD)
                    src = fidx
                else:
                    src = ridx
                cps = [pltpu.async_copy(tbl.at[src.at[j]], gath.at[j], gsem)
                       for j in range(_NCH)]
                for cp in cps:
                    cp.wait()

                @pl.loop(0, _CH, step=16)
                def _mul(i):
                    for j in range(_NCH):
                        val[j, pl.ds(i, 16)] = (eww[j, pl.ds(i, 16)]
                                                * gath[j, pl.ds(i, 16)])
            else:
                @pl.loop(0, _CH, step=16)
                def _cpv(i):
                    for j in range(_NCH):
                        val[j, pl.ds(i, 16)] = eww[j, pl.ds(i, 16)]
            if k > 1:
                @pl.loop(0, _CH, step=16)
                def _ofc(i):
                    for j in range(_NCH):
                        fidx[j, pl.ds(i, 16)] = (
                            cidx[j, pl.ds(i, 16)] + f * _NPAD)
                dst = fidx
            else:
                dst = cidx
            scs = [pltpu.async_copy(val.at[j], acc.at[dst.at[j]], ssem,
                                    add=True)
                   for j in range(_NCH)]
            for cp in scs:
                cp.wait()

    plsc.subcore_barrier()
    osl = pl.ds((core * 16 + tid) * k * _RPT, k * _RPT)
    pltpu.sync_copy(acc.at[sl], out.at[osl])


def _make_seg(k, gather):
    tshape = (k * _NPAD,)
    scratch = [
        pltpu.VMEM((_NCH, _CH), jnp.int32),
        pltpu.VMEM((_NCH, _CH), jnp.int32),
        pltpu.VMEM((_NCH, _CH), jnp.float32),
        pltpu.VMEM((_NCH, _CH), jnp.int32),
        pltpu.VMEM((_NCH, _CH), jnp.float32),
        pltpu.VMEM((_NCH, _CH), jnp.float32),
        pltpu.VMEM_SHARED(tshape, jnp.float32),
        pltpu.VMEM_SHARED(tshape, jnp.float32),
        pltpu.SemaphoreType.DMA,
        pltpu.SemaphoreType.DMA,
    ]
    kern = pl.kernel(
        functools.partial(_seg_body, k, gather),
        out_type=jax.ShapeDtypeStruct((2 * k * _NPAD,), jnp.float32),
        mesh=_mesh,
        scratch_types=scratch,
        compiler_params=pltpu.CompilerParams(needs_layout_passes=False),
    )

    def run(table, row2d, col2d, ew2d):
        zeros = jnp.zeros(tshape, jnp.float32)
        return kern(table, row2d, col2d, ew2d, zeros).reshape(2, k, _NPAD)

    return run


_seg_passes = {}


def _seg(k, gather=True):
    key = (k, gather)
    if key not in _seg_passes:
        _seg_passes[key] = _make_seg(k, gather)
    return _seg_passes[key]


def _inorm(x, eps=1e-5):
    mean = jnp.mean(x, axis=0, keepdims=True)
    var = jnp.var(x, axis=0, keepdims=True)
    return (x - mean) / jnp.sqrt(var + eps)


def _edge_mlp(src, dest, ea, W1, b1, g, be, W2, b2, eps=1e-5):
    h = jnp.concatenate([src, dest, ea], axis=1)
    h = jax.nn.relu(h @ W1 + b1)
    mu = jnp.mean(h, axis=-1, keepdims=True)
    var = jnp.var(h, axis=-1, keepdims=True)
    h = (h - mu) / jnp.sqrt(var + eps) * g + be
    return h @ W2 + b2


def _mlp_body(xs_ref, w1_ref, b1_ref, w2_ref, b2_ref, w3_ref, b3_ref, out_ref):
    h = jnp.maximum(xs_ref[...] @ w1_ref[...] + b1_ref[...], 0.0)
    h = jnp.maximum(h @ w2_ref[...] + b2_ref[...], 0.0)
    out_ref[...] = jnp.maximum(h @ w3_ref[...] + b3_ref[...], 0.0)


def _node_mlp(xs, lin1_W, lin1_b, lin2_W, lin2_b, lin3_W, lin3_b):
    n = xs.shape[0]
    blk = 2000
    grid = n // blk
    full = lambda s: pl.BlockSpec(s, lambda i: (0,) * len(s))
    return pl.pallas_call(
        _mlp_body,
        grid=(grid,),
        in_specs=[
            pl.BlockSpec((blk, 85), lambda i: (i, 0)),
            full((85, 40)), full((40,)),
            full((40, 16)), full((16,)),
            full((16, 1)), full((1,)),
        ],
        out_specs=pl.BlockSpec((blk, 1), lambda i: (i, 0)),
        out_shape=jax.ShapeDtypeStruct((n, 1), jnp.float32),
    )(xs, lin1_W, lin1_b, lin2_W, lin2_b, lin3_W, lin3_b)


def kernel(x, edge_index, edge_attr, nc1_W, nc1_b, nc2_W, nc2_b, nc3_W, nc3_b,
           nc4_W, nc4_b, lin1_W, lin1_b, lin2_W, lin2_b, lin3_W, lin3_b,
           ec1_W1, ec1_b1, ec1_g, ec1_be, ec1_W2, ec1_b2,
           ec2_W1, ec2_b1, ec2_g, ec2_be, ec2_W2, ec2_b2):
    x = x.reshape(-1, 1)
    row = edge_index[0]
    col = edge_index[1]
    ew = edge_attr.reshape(-1)

    epad = _EPAD - _E
    row2d = jnp.pad(row, (0, epad)).reshape(-1, _CH)
    col2d = jnp.pad(col, (0, epad)).reshape(-1, _CH)
    ew2d = jnp.pad(ew, (0, epad)).reshape(-1, _CH)

    degp = _seg(1, gather=False)(jnp.zeros((_NPAD,), jnp.float32),
                                 col2d, row2d, ew2d)
    deg = (degp[:, 0].sum(0))[:_N]
    dis = jnp.where(deg > 0, jax.lax.rsqrt(deg), 0.0)
    disc = dis[:, None]

    def lmul(v):
        c = v.shape[1]
        u = jnp.pad((disc * v).T, ((0, 0), (0, _NPAD - _N))).reshape(-1)
        p = _seg(c)(u, row2d, col2d, ew2d)
        a = (p[0] + p[1])[:, :_N].T
        return -disc * a

    def cheb(xin, W, b):
        Tx1 = lmul(xin)
        Tx2 = 2.0 * lmul(Tx1) - xin
        return xin @ W[0] + Tx1 @ W[1] + Tx2 @ W[2] + b

    x1 = jax.nn.relu(cheb(_inorm(x), nc1_W, nc1_b))
    x2 = jax.nn.relu(cheb(_inorm(x1), nc2_W, nc2_b))
    x3 = jax.nn.relu(cheb(_inorm(x2), nc3_W, nc3_b))

    xin4 = _inorm(x3)
    u12 = xin4 @ jnp.concatenate([nc4_W[1], nc4_W[2]], axis=1)
    t = lmul(u12)
    lm2 = lmul(t[:, 1:2])
    x4 = jax.nn.relu(xin4 @ nc4_W[0] + t[:, 0:1] + 2.0 * lm2
                     - u12[:, 1:2] + nc4_b)

    xs = jnp.concatenate([x1, x2, x3, x4], axis=1)
    xo = _node_mlp(xs, lin1_W, lin1_b, lin2_W, lin2_b, lin3_W, lin3_b)
    ea = jax.nn.relu(_edge_mlp(xo[row], xo[col], edge_attr,
                               ec1_W1, ec1_b1, ec1_g, ec1_be, ec1_W2, ec1_b2))
    ea = jax.nn.relu(_edge_mlp(xo[row], xo[col], ea,
                               ec2_W1, ec2_b1, ec2_g, ec2_be, ec2_W2, ec2_b2))
    return (xo, ea)
